# baseline (speedup 1.0000x reference)
import functools

import jax
import jax.numpy as jnp
from jax import lax
from jax.experimental import pallas as pl
from jax.experimental.pallas import tpu as pltpu
from jax.experimental.pallas import tpu_sc as plsc

SEQ = 2048
EXPERTS = 16
HIDDEN = 2048
TOPK = 2
CAPACITY = 256
BLK = 256
NBLK = SEQ // BLK

NWORKERS = 32
TOK_PER_W = SEQ // NWORKERS
CHUNK = 8
NCHUNK = TOK_PER_W // CHUNK
ROW = EXPERTS * CAPACITY
CHUNK_WORDS = CHUNK * ROW


def _inclusive_cumsum_rows(x):
    n = x.shape[0]
    d = 1
    while d < n:
        shifted = jnp.concatenate(
            [jnp.zeros((d, x.shape[1]), x.dtype), x[:-d, :]], axis=0
        )
        x = x + shifted
        d *= 2
    return x


def _routing_body(hs_ref, wg_ref, rp_ref, p0_ref, p1_ref, c0_ref, cnt_ref,
                  laux_ref, rate_ref, offs0, offs1, sumg):
    i = pl.program_id(0)

    @pl.when(i == 0)
    def _init():
        offs0[...] = jnp.zeros_like(offs0)
        offs1[...] = jnp.zeros_like(offs1)
        sumg[...] = jnp.zeros_like(sumg)

    x = hs_ref[...]
    w = wg_ref[...]
    logits = jax.lax.dot_general(
        x, w, (((1,), (1,)), ((), ())), preferred_element_type=jnp.float32
    )

    m = jnp.max(logits, axis=1, keepdims=True)
    ex = jnp.exp(logits - m)
    g = ex / jnp.sum(ex, axis=1, keepdims=True)

    iota = jax.lax.broadcasted_iota(jnp.int32, (BLK, EXPERTS), 1)
    v0 = jnp.max(g, axis=1, keepdims=True)
    idx0 = jnp.min(jnp.where(g == v0, iota, EXPERTS), axis=1, keepdims=True)
    m0 = iota == idx0
    g_ex = jnp.where(m0, -jnp.inf, g)
    v1 = jnp.max(g_ex, axis=1, keepdims=True)
    idx1 = jnp.min(jnp.where(g_ex == v1, iota, EXPERTS), axis=1, keepdims=True)
    m1 = iota == idx1

    gates_s = jnp.maximum(v0 + v1, jnp.finfo(jnp.float32).eps)
    rp_ref[...] = g / gates_s

    m0f = m0.astype(jnp.float32)
    m1f = m1.astype(jnp.float32)
    inc0 = _inclusive_cumsum_rows(m0f)
    inc1 = _inclusive_cumsum_rows(m1f)
    exc0 = inc0 - m0f
    exc1 = inc1 - m1f

    p0_ref[...] = jnp.where(m0, offs0[...] + exc0, -1.0)
    p1_ref[...] = jnp.where(m1, offs1[...] + exc1, -1.0)

    offs0[...] = offs0[...] + inc0[BLK - 1 : BLK, :]
    offs1[...] = offs1[...] + inc1[BLK - 1 : BLK, :]
    sumg[...] = sumg[...] + jnp.sum(g, axis=0, keepdims=True)
    c0_ref[...] = offs0[...]

    @pl.when(i == NBLK - 1)
    def _finish():
        ctot = offs0[...] + offs1[...]
        cnt_ref[...] = ctot.astype(jnp.int32)
        inv_s = 1.0 / SEQ
        laux = (EXPERTS * EXPERTS) * jnp.mean(
            (ctot * inv_s) * (sumg[...] * inv_s)
        )
        laux_ref[0, 0] = laux
        rate_ref[0, 0] = jnp.sum(jnp.minimum(ctot, float(CAPACITY))) / (
            SEQ * TOPK
        )


def _dispatch_body(p0_ref, p1_ref, c0_ref, disp_ref):
    p0 = p0_ref[...]
    p1p = p1_ref[...]
    c0 = c0_ref[...]

    p1 = jnp.where(p1p >= 0.0, p1p + c0, -1.0)
    tp = jnp.maximum(p0, p1)
    valid = jnp.logical_and(tp >= 0.0, tp < float(CAPACITY))
    tpc = jnp.where(valid, tp, -1.0).astype(jnp.int32)

    cap_iota = jax.lax.broadcasted_iota(
        jnp.int32, (BLK, EXPERTS, CAPACITY), 2
    )
    disp_ref[...] = tpc[:, :, None] == cap_iota


def _combine_sc_body(rp_hbm, p0_hbm, p1_hbm, c0_hbm, out_hbm,
                     rp_v, p0_v, p1_v, c0_v, buf0, buf1, sem0, sem1):
    wid = lax.axis_index("s") * 2 + lax.axis_index("c")
    base = wid * TOK_PER_W

    pltpu.sync_copy(rp_hbm.at[pl.ds(base, TOK_PER_W)], rp_v)
    pltpu.sync_copy(p0_hbm.at[pl.ds(base, TOK_PER_W)], p0_v)
    pltpu.sync_copy(p1_hbm.at[pl.ds(base, TOK_PER_W)], p1_v)
    pltpu.sync_copy(c0_hbm, c0_v)

    lanes = lax.iota(jnp.int32, 16)
    zf = jnp.zeros((16,), jnp.float32)

    def _zero_body(i, _):
        idx = lanes + i * 16
        plsc.store_scatter(buf0, [idx], zf)
        plsc.store_scatter(buf1, [idx], zf)
        return _

    lax.fori_loop(0, CHUNK_WORDS // 16, _zero_body, 0)

    c0r = c0_v[...]
    bufs = [buf0, buf1]
    sems = [sem0, sem1]
    copies = [None, None]
    stale = [None, None]

    for cidx in range(NCHUNK):
        b = cidx % 2
        buf = bufs[b]
        if copies[b] is not None:
            copies[b].wait()
            for idx, msk in stale[b]:
                plsc.store_scatter(buf, [idx], zf, mask=msk)
        entries = []
        for t in range(CHUNK):
            tok = cidx * CHUNK + t
            p0r = p0_v[tok]
            p1r = p1_v[tok]
            rpr = rp_v[tok]
            p1f = jnp.where(p1r >= 0.0, p1r + c0r, -1.0)
            tp = jnp.maximum(p0r, p1f)
            valid = jnp.logical_and(tp >= 0.0, tp < float(CAPACITY))
            pos = jnp.minimum(jnp.maximum(tp, 0.0), float(CAPACITY - 1))
            idx = t * ROW + lanes * CAPACITY + pos.astype(jnp.int32)
            plsc.store_scatter(buf, [idx], rpr, mask=valid)
            entries.append((idx, valid))
        stale[b] = entries
        copies[b] = pltpu.async_copy(
            buf,
            out_hbm.at[pl.ds((base + cidx * CHUNK) * ROW, CHUNK_WORDS)],
            sems[b],
        )
    for b in range(2):
        if copies[b] is not None:
            copies[b].wait()


_combine_sc = functools.partial(
    pl.kernel,
    out_type=jax.ShapeDtypeStruct((SEQ * ROW,), jnp.float32),
    mesh=plsc.VectorSubcoreMesh(core_axis_name="c", subcore_axis_name="s"),
    scratch_types=[
        pltpu.VMEM((TOK_PER_W, EXPERTS), jnp.float32),
        pltpu.VMEM((TOK_PER_W, EXPERTS), jnp.float32),
        pltpu.VMEM((TOK_PER_W, EXPERTS), jnp.float32),
        pltpu.VMEM((EXPERTS,), jnp.float32),
        pltpu.VMEM((CHUNK_WORDS,), jnp.float32),
        pltpu.VMEM((CHUNK_WORDS,), jnp.float32),
        pltpu.SemaphoreType.DMA,
        pltpu.SemaphoreType.DMA,
    ],
    compiler_params=pltpu.CompilerParams(needs_layout_passes=False),
)(_combine_sc_body)


@jax.jit
def _run(hs, wg):
    meta_spec = pl.BlockSpec((BLK, EXPERTS), lambda i: (i, 0))
    vec_spec = pl.BlockSpec((1, EXPERTS), lambda i: (0, 0))
    smem_spec = pl.BlockSpec(memory_space=pltpu.SMEM)

    rp, p0, p1, c0, cnt, laux, rate = pl.pallas_call(
        _routing_body,
        grid=(NBLK,),
        in_specs=[
            pl.BlockSpec((BLK, HIDDEN), lambda i: (i, 0)),
            pl.BlockSpec((EXPERTS, HIDDEN), lambda i: (0, 0)),
        ],
        out_specs=[meta_spec, meta_spec, meta_spec, vec_spec, vec_spec,
                   smem_spec, smem_spec],
        out_shape=[
            jax.ShapeDtypeStruct((SEQ, EXPERTS), jnp.float32),
            jax.ShapeDtypeStruct((SEQ, EXPERTS), jnp.float32),
            jax.ShapeDtypeStruct((SEQ, EXPERTS), jnp.float32),
            jax.ShapeDtypeStruct((1, EXPERTS), jnp.float32),
            jax.ShapeDtypeStruct((1, EXPERTS), jnp.int32),
            jax.ShapeDtypeStruct((1, 1), jnp.float32),
            jax.ShapeDtypeStruct((1, 1), jnp.float32),
        ],
        scratch_shapes=[
            pltpu.VMEM((1, EXPERTS), jnp.float32),
            pltpu.VMEM((1, EXPERTS), jnp.float32),
            pltpu.VMEM((1, EXPERTS), jnp.float32),
        ],
    )(hs, wg)

    comb_flat = _combine_sc(rp, p0, p1, c0.reshape(EXPERTS))

    disp = pl.pallas_call(
        _dispatch_body,
        grid=(NBLK,),
        in_specs=[meta_spec, meta_spec, vec_spec],
        out_specs=[
            pl.BlockSpec((BLK, EXPERTS, CAPACITY), lambda i: (i, 0, 0)),
        ],
        out_shape=[
            jax.ShapeDtypeStruct((SEQ, EXPERTS, CAPACITY), jnp.bool_),
        ],
    )(p0, p1, c0)[0]

    return (
        laux.reshape(()),
        rate.reshape(()),
        comb_flat.reshape(SEQ, EXPERTS, CAPACITY),
        disp,
        cnt.reshape(EXPERTS),
    )


def kernel(hidden_states, wg_weight):
    hs = hidden_states.reshape(-1, HIDDEN).astype(jnp.float32)
    return _run(hs, wg_weight)


# speedup vs baseline: 1.8243x; 1.8243x over previous
import functools

import jax
import jax.numpy as jnp
from jax import lax
from jax.experimental import pallas as pl
from jax.experimental.pallas import tpu as pltpu
from jax.experimental.pallas import tpu_sc as plsc

SEQ = 2048
EXPERTS = 16
HIDDEN = 2048
TOPK = 2
CAPACITY = 256
BLK = 256
NBLK = SEQ // BLK

NWORKERS = 32
TOK_PER_W = SEQ // NWORKERS
CHUNK = 8
NCHUNK = TOK_PER_W // CHUNK
ROW = EXPERTS * CAPACITY
CHUNK_WORDS = CHUNK * ROW


def _inclusive_cumsum_rows(x):
    n = x.shape[0]
    d = 1
    while d < n:
        shifted = jnp.concatenate(
            [jnp.zeros((d, x.shape[1]), x.dtype), x[:-d, :]], axis=0
        )
        x = x + shifted
        d *= 2
    return x


def _routing_body(hs_ref, wg_ref, tril_ref, rp_ref, p0_ref, p1_ref, c0_ref,
                  cnt_ref, laux_ref, rate_ref, offs0, offs1, sumg):
    i = pl.program_id(0)

    @pl.when(i == 0)
    def _init():
        offs0[...] = jnp.zeros_like(offs0)
        offs1[...] = jnp.zeros_like(offs1)
        sumg[...] = jnp.zeros_like(sumg)

    x = hs_ref[...]
    w = wg_ref[...]
    logits = jax.lax.dot_general(
        x, w, (((1,), (1,)), ((), ())), preferred_element_type=jnp.float32
    )

    m = jnp.max(logits, axis=1, keepdims=True)
    ex = jnp.exp(logits - m)
    g = ex / jnp.sum(ex, axis=1, keepdims=True)

    iota = jax.lax.broadcasted_iota(jnp.int32, (BLK, EXPERTS), 1)
    v0 = jnp.max(g, axis=1, keepdims=True)
    idx0 = jnp.min(jnp.where(g == v0, iota, EXPERTS), axis=1, keepdims=True)
    m0 = iota == idx0
    g_ex = jnp.where(m0, -jnp.inf, g)
    v1 = jnp.max(g_ex, axis=1, keepdims=True)
    idx1 = jnp.min(jnp.where(g_ex == v1, iota, EXPERTS), axis=1, keepdims=True)
    m1 = iota == idx1

    gates_s = jnp.maximum(v0 + v1, jnp.finfo(jnp.float32).eps)
    rp_ref[...] = g / gates_s

    m0f = m0.astype(jnp.float32)
    m1f = m1.astype(jnp.float32)
    tril = tril_ref[...]
    exc0 = jax.lax.dot_general(
        tril, m0f, (((1,), (0,)), ((), ())), preferred_element_type=jnp.float32
    )
    exc1 = jax.lax.dot_general(
        tril, m1f, (((1,), (0,)), ((), ())), preferred_element_type=jnp.float32
    )

    p0_ref[...] = jnp.where(m0, offs0[...] + exc0, -1.0)
    p1_ref[...] = jnp.where(m1, offs1[...] + exc1, -1.0)

    tot0 = exc0[BLK - 1 : BLK, :] + m0f[BLK - 1 : BLK, :]
    tot1 = exc1[BLK - 1 : BLK, :] + m1f[BLK - 1 : BLK, :]
    offs0[...] = offs0[...] + tot0
    offs1[...] = offs1[...] + tot1
    sumg[...] = sumg[...] + jnp.sum(g, axis=0, keepdims=True)
    c0_ref[...] = offs0[...]

    @pl.when(i == NBLK - 1)
    def _finish():
        ctot = offs0[...] + offs1[...]
        cnt_ref[...] = ctot.astype(jnp.int32)
        inv_s = 1.0 / SEQ
        laux = (EXPERTS * EXPERTS) * jnp.mean(
            (ctot * inv_s) * (sumg[...] * inv_s)
        )
        laux_ref[0, 0] = laux
        rate_ref[0, 0] = jnp.sum(jnp.minimum(ctot, float(CAPACITY))) / (
            SEQ * TOPK
        )


def _dispatch_body(p0_ref, p1_ref, c0_ref, disp_ref):
    p0 = p0_ref[...]
    p1p = p1_ref[...]
    c0 = c0_ref[...]

    p1 = jnp.where(p1p >= 0.0, p1p + c0, -1.0)
    tp = jnp.maximum(p0, p1)
    valid = jnp.logical_and(tp >= 0.0, tp < float(CAPACITY))
    tpc = jnp.where(valid, tp, -1.0).astype(jnp.int32)

    cap_iota = jax.lax.broadcasted_iota(
        jnp.int32, (BLK, EXPERTS, CAPACITY), 2
    )
    disp_ref[...] = (tpc[:, :, None] == cap_iota).astype(jnp.int8)


def _combine_sc_body(rp_hbm, p0_hbm, p1_hbm, c0_hbm, out_hbm,
                     rp_v, p0_v, p1_v, c0_v, buf0, buf1, sem0, sem1):
    wid = lax.axis_index("s") * 2 + lax.axis_index("c")
    base = wid * TOK_PER_W

    pltpu.sync_copy(rp_hbm.at[pl.ds(base, TOK_PER_W)], rp_v)
    pltpu.sync_copy(p0_hbm.at[pl.ds(base, TOK_PER_W)], p0_v)
    pltpu.sync_copy(p1_hbm.at[pl.ds(base, TOK_PER_W)], p1_v)
    pltpu.sync_copy(c0_hbm, c0_v)

    lanes = lax.iota(jnp.int32, 16)
    zf = jnp.zeros((16,), jnp.float32)

    def _zero_body(i, _):
        flat = i * 64
        for u in range(4):
            f = lanes + (flat + u * 16)
            it = jax.lax.shift_right_logical(f, 12)
            ie = jax.lax.shift_right_logical(f, 8) & 15
            ic = f & 255
            plsc.store_scatter(buf0, [it, ie, ic], zf)
            plsc.store_scatter(buf1, [it, ie, ic], zf)
        return _

    lax.fori_loop(0, CHUNK_WORDS // 64, _zero_body, 0)

    c0r = c0_v[...]
    bufs = [buf0, buf1]
    sems = [sem0, sem1]
    copies = [None, None]
    stale = [None, None]

    for cidx in range(NCHUNK):
        b = cidx % 2
        buf = bufs[b]
        if copies[b] is not None:
            copies[b].wait()
            for it, ic, msk in stale[b]:
                plsc.store_scatter(buf, [it, lanes, ic], zf, mask=msk)
        entries = []
        for t in range(CHUNK):
            tok = cidx * CHUNK + t
            p0r = p0_v[tok]
            p1r = p1_v[tok]
            rpr = rp_v[tok]
            p1f = jnp.where(p1r >= 0.0, p1r + c0r, -1.0)
            tp = jnp.maximum(p0r, p1f)
            valid = jnp.logical_and(tp >= 0.0, tp < float(CAPACITY))
            pos = jnp.minimum(jnp.maximum(tp, 0.0), float(CAPACITY - 1))
            it = jnp.full((16,), t, jnp.int32)
            ic = pos.astype(jnp.int32)
            plsc.store_scatter(buf, [it, lanes, ic], rpr, mask=valid)
            entries.append((it, ic, valid))
        stale[b] = entries
        copies[b] = pltpu.async_copy(
            buf,
            out_hbm.at[pl.ds(base + cidx * CHUNK, CHUNK)],
            sems[b],
        )
    for b in range(2):
        if copies[b] is not None:
            copies[b].wait()


_combine_sc = functools.partial(
    pl.kernel,
    out_type=jax.ShapeDtypeStruct((SEQ, EXPERTS, CAPACITY), jnp.float32),
    mesh=plsc.VectorSubcoreMesh(core_axis_name="c", subcore_axis_name="s"),
    scratch_types=[
        pltpu.VMEM((TOK_PER_W, EXPERTS), jnp.float32),
        pltpu.VMEM((TOK_PER_W, EXPERTS), jnp.float32),
        pltpu.VMEM((TOK_PER_W, EXPERTS), jnp.float32),
        pltpu.VMEM((EXPERTS,), jnp.float32),
        pltpu.VMEM((CHUNK, EXPERTS, CAPACITY), jnp.float32),
        pltpu.VMEM((CHUNK, EXPERTS, CAPACITY), jnp.float32),
        pltpu.SemaphoreType.DMA,
        pltpu.SemaphoreType.DMA,
    ],
    compiler_params=pltpu.CompilerParams(needs_layout_passes=False),
)(_combine_sc_body)


@jax.jit
def _run(hs, wg):
    meta_spec = pl.BlockSpec((BLK, EXPERTS), lambda i: (i, 0))
    vec_spec = pl.BlockSpec((1, EXPERTS), lambda i: (0, 0))
    smem_spec = pl.BlockSpec(memory_space=pltpu.SMEM)

    row_iota = jax.lax.broadcasted_iota(jnp.int32, (BLK, BLK), 0)
    col_iota = jax.lax.broadcasted_iota(jnp.int32, (BLK, BLK), 1)
    tril = (col_iota < row_iota).astype(jnp.float32)

    rp, p0, p1, c0, cnt, laux, rate = pl.pallas_call(
        _routing_body,
        grid=(NBLK,),
        in_specs=[
            pl.BlockSpec((BLK, HIDDEN), lambda i: (i, 0)),
            pl.BlockSpec((EXPERTS, HIDDEN), lambda i: (0, 0)),
            pl.BlockSpec((BLK, BLK), lambda i: (0, 0)),
        ],
        out_specs=[meta_spec, meta_spec, meta_spec, vec_spec, vec_spec,
                   smem_spec, smem_spec],
        out_shape=[
            jax.ShapeDtypeStruct((SEQ, EXPERTS), jnp.float32),
            jax.ShapeDtypeStruct((SEQ, EXPERTS), jnp.float32),
            jax.ShapeDtypeStruct((SEQ, EXPERTS), jnp.float32),
            jax.ShapeDtypeStruct((1, EXPERTS), jnp.float32),
            jax.ShapeDtypeStruct((1, EXPERTS), jnp.int32),
            jax.ShapeDtypeStruct((1, 1), jnp.float32),
            jax.ShapeDtypeStruct((1, 1), jnp.float32),
        ],
        scratch_shapes=[
            pltpu.VMEM((1, EXPERTS), jnp.float32),
            pltpu.VMEM((1, EXPERTS), jnp.float32),
            pltpu.VMEM((1, EXPERTS), jnp.float32),
        ],
    )(hs, wg, tril)

    comb = _combine_sc(rp, p0, p1, c0.reshape(EXPERTS))

    disp8 = pl.pallas_call(
        _dispatch_body,
        grid=(NBLK,),
        in_specs=[meta_spec, meta_spec, vec_spec],
        out_specs=[
            pl.BlockSpec((BLK, EXPERTS, CAPACITY), lambda i: (i, 0, 0)),
        ],
        out_shape=[
            jax.ShapeDtypeStruct((SEQ, EXPERTS, CAPACITY), jnp.int8),
        ],
    )(p0, p1, c0)[0]

    return (
        laux.reshape(()),
        rate.reshape(()),
        comb,
        disp8.astype(jnp.bool_),
        cnt.reshape(EXPERTS),
    )


def kernel(hidden_states, wg_weight):
    hs = hidden_states.reshape(-1, HIDDEN).astype(jnp.float32)
    return _run(hs, wg_weight)


# speedup vs baseline: 1.9138x; 1.0491x over previous
import functools

import jax
import jax.numpy as jnp
from jax import lax
from jax.experimental import pallas as pl
from jax.experimental.pallas import tpu as pltpu
from jax.experimental.pallas import tpu_sc as plsc

SEQ = 2048
EXPERTS = 16
HIDDEN = 2048
TOPK = 2
CAPACITY = 256
BLK = 512
NBLK = SEQ // BLK

NWORKERS = 32
TOK_PER_W = SEQ // NWORKERS
CHUNK = 8
NCHUNK = TOK_PER_W // CHUNK
ROW = EXPERTS * CAPACITY
CHUNK_WORDS = CHUNK * ROW


def _inclusive_cumsum_rows(x):
    n = x.shape[0]
    d = 1
    while d < n:
        shifted = jnp.concatenate(
            [jnp.zeros((d, x.shape[1]), x.dtype), x[:-d, :]], axis=0
        )
        x = x + shifted
        d *= 2
    return x


def _routing_body(hs_ref, wg_ref, tril_ref, rp_ref, p0_ref, p1_ref, c0_ref,
                  cnt_ref, laux_ref, rate_ref, offs0, offs1, sumg):
    i = pl.program_id(0)

    @pl.when(i == 0)
    def _init():
        offs0[...] = jnp.zeros_like(offs0)
        offs1[...] = jnp.zeros_like(offs1)
        sumg[...] = jnp.zeros_like(sumg)

    x = hs_ref[...]
    w = wg_ref[...]
    logits = jax.lax.dot_general(
        x, w, (((1,), (1,)), ((), ())), preferred_element_type=jnp.float32
    )

    m = jnp.max(logits, axis=1, keepdims=True)
    ex = jnp.exp(logits - m)
    g = ex / jnp.sum(ex, axis=1, keepdims=True)

    iota = jax.lax.broadcasted_iota(jnp.int32, (BLK, EXPERTS), 1)
    v0 = jnp.max(g, axis=1, keepdims=True)
    idx0 = jnp.min(jnp.where(g == v0, iota, EXPERTS), axis=1, keepdims=True)
    m0 = iota == idx0
    g_ex = jnp.where(m0, -jnp.inf, g)
    v1 = jnp.max(g_ex, axis=1, keepdims=True)
    idx1 = jnp.min(jnp.where(g_ex == v1, iota, EXPERTS), axis=1, keepdims=True)
    m1 = iota == idx1

    gates_s = jnp.maximum(v0 + v1, jnp.finfo(jnp.float32).eps)
    rp_ref[...] = g / gates_s

    m0f = m0.astype(jnp.float32)
    m1f = m1.astype(jnp.float32)
    tril = tril_ref[...]
    exc0 = jax.lax.dot_general(
        tril, m0f, (((1,), (0,)), ((), ())), preferred_element_type=jnp.float32
    )
    exc1 = jax.lax.dot_general(
        tril, m1f, (((1,), (0,)), ((), ())), preferred_element_type=jnp.float32
    )

    p0_ref[...] = jnp.where(m0, offs0[...] + exc0, -1.0)
    p1_ref[...] = jnp.where(m1, offs1[...] + exc1, -1.0)

    tot0 = exc0[BLK - 1 : BLK, :] + m0f[BLK - 1 : BLK, :]
    tot1 = exc1[BLK - 1 : BLK, :] + m1f[BLK - 1 : BLK, :]
    offs0[...] = offs0[...] + tot0
    offs1[...] = offs1[...] + tot1
    sumg[...] = sumg[...] + jnp.sum(g, axis=0, keepdims=True)
    c0_ref[...] = offs0[...]

    @pl.when(i == NBLK - 1)
    def _finish():
        ctot = offs0[...] + offs1[...]
        cnt_ref[...] = ctot.astype(jnp.int32)
        inv_s = 1.0 / SEQ
        laux = (EXPERTS * EXPERTS) * jnp.mean(
            (ctot * inv_s) * (sumg[...] * inv_s)
        )
        laux_ref[0, 0] = laux
        rate_ref[0, 0] = jnp.sum(jnp.minimum(ctot, float(CAPACITY))) / (
            SEQ * TOPK
        )


def _dispatch_body(p0_ref, p1_ref, c0_ref, disp_ref):
    p0 = p0_ref[...]
    p1p = p1_ref[...]
    c0 = c0_ref[...]

    p1 = jnp.where(p1p >= 0.0, p1p + c0, -1.0)
    tp = jnp.maximum(p0, p1)
    valid = jnp.logical_and(tp >= 0.0, tp < float(CAPACITY))
    tpc = jnp.where(valid, tp, -1.0).astype(jnp.int32)

    cap_iota = jax.lax.broadcasted_iota(
        jnp.int32, (BLK, EXPERTS, CAPACITY), 2
    )
    disp_ref[...] = (tpc[:, :, None] == cap_iota).astype(jnp.int8)


def _combine_sc_body(rp_hbm, p0_hbm, p1_hbm, c0_hbm, out_hbm,
                     rp_v, p0_v, p1_v, c0_v, buf0, buf1, buf2,
                     sem0, sem1, sem2):
    wid = lax.axis_index("s") * 2 + lax.axis_index("c")
    base = wid * TOK_PER_W

    pltpu.sync_copy(rp_hbm.at[pl.ds(base, TOK_PER_W)], rp_v)
    pltpu.sync_copy(p0_hbm.at[pl.ds(base, TOK_PER_W)], p0_v)
    pltpu.sync_copy(p1_hbm.at[pl.ds(base, TOK_PER_W)], p1_v)
    pltpu.sync_copy(c0_hbm, c0_v)

    lanes = lax.iota(jnp.int32, 16)
    zf = jnp.zeros((16,), jnp.float32)

    def _zero_body(i, _):
        flat = i * 64
        for u in range(4):
            f = lanes + (flat + u * 16)
            it = jax.lax.shift_right_logical(f, 12)
            ie = jax.lax.shift_right_logical(f, 8) & 15
            ic = f & 255
            plsc.store_scatter(buf0, [it, ie, ic], zf)
            plsc.store_scatter(buf1, [it, ie, ic], zf)
            plsc.store_scatter(buf2, [it, ie, ic], zf)
        return _

    lax.fori_loop(0, CHUNK_WORDS // 64, _zero_body, 0)

    c0r = c0_v[...]
    bufs = [buf0, buf1, buf2]
    sems = [sem0, sem1, sem2]
    copies = [None, None, None]
    stale = [None, None, None]

    for cidx in range(NCHUNK):
        b = cidx % 3
        buf = bufs[b]
        if copies[b] is not None:
            copies[b].wait()
            for it, ic, msk in stale[b]:
                plsc.store_scatter(buf, [it, lanes, ic], zf, mask=msk)
        entries = []
        for t in range(CHUNK):
            tok = cidx * CHUNK + t
            p0r = p0_v[tok]
            p1r = p1_v[tok]
            rpr = rp_v[tok]
            p1f = jnp.where(p1r >= 0.0, p1r + c0r, -1.0)
            tp = jnp.maximum(p0r, p1f)
            valid = jnp.logical_and(tp >= 0.0, tp < float(CAPACITY))
            pos = jnp.minimum(jnp.maximum(tp, 0.0), float(CAPACITY - 1))
            it = jnp.full((16,), t, jnp.int32)
            ic = pos.astype(jnp.int32)
            plsc.store_scatter(buf, [it, lanes, ic], rpr, mask=valid)
            entries.append((it, ic, valid))
        stale[b] = entries
        copies[b] = pltpu.async_copy(
            buf,
            out_hbm.at[pl.ds(base + cidx * CHUNK, CHUNK)],
            sems[b],
        )
    for b in range(3):
        if copies[b] is not None:
            copies[b].wait()


_combine_sc = functools.partial(
    pl.kernel,
    out_type=jax.ShapeDtypeStruct((SEQ, EXPERTS, CAPACITY), jnp.float32),
    mesh=plsc.VectorSubcoreMesh(core_axis_name="c", subcore_axis_name="s"),
    scratch_types=[
        pltpu.VMEM((TOK_PER_W, EXPERTS), jnp.float32),
        pltpu.VMEM((TOK_PER_W, EXPERTS), jnp.float32),
        pltpu.VMEM((TOK_PER_W, EXPERTS), jnp.float32),
        pltpu.VMEM((EXPERTS,), jnp.float32),
        pltpu.VMEM((CHUNK, EXPERTS, CAPACITY), jnp.float32),
        pltpu.VMEM((CHUNK, EXPERTS, CAPACITY), jnp.float32),
        pltpu.VMEM((CHUNK, EXPERTS, CAPACITY), jnp.float32),
        pltpu.SemaphoreType.DMA,
        pltpu.SemaphoreType.DMA,
        pltpu.SemaphoreType.DMA,
    ],
    compiler_params=pltpu.CompilerParams(
        needs_layout_passes=False, skip_device_barrier=True
    ),
)(_combine_sc_body)


@jax.jit
def _run(hs, wg):
    meta_spec = pl.BlockSpec((BLK, EXPERTS), lambda i: (i, 0))
    vec_spec = pl.BlockSpec((1, EXPERTS), lambda i: (0, 0))
    smem_spec = pl.BlockSpec(memory_space=pltpu.SMEM)

    row_iota = jax.lax.broadcasted_iota(jnp.int32, (BLK, BLK), 0)
    col_iota = jax.lax.broadcasted_iota(jnp.int32, (BLK, BLK), 1)
    tril = (col_iota < row_iota).astype(jnp.float32)

    rp, p0, p1, c0, cnt, laux, rate = pl.pallas_call(
        _routing_body,
        grid=(NBLK,),
        in_specs=[
            pl.BlockSpec((BLK, HIDDEN), lambda i: (i, 0)),
            pl.BlockSpec((EXPERTS, HIDDEN), lambda i: (0, 0)),
            pl.BlockSpec((BLK, BLK), lambda i: (0, 0)),
        ],
        out_specs=[meta_spec, meta_spec, meta_spec, vec_spec, vec_spec,
                   smem_spec, smem_spec],
        out_shape=[
            jax.ShapeDtypeStruct((SEQ, EXPERTS), jnp.float32),
            jax.ShapeDtypeStruct((SEQ, EXPERTS), jnp.float32),
            jax.ShapeDtypeStruct((SEQ, EXPERTS), jnp.float32),
            jax.ShapeDtypeStruct((1, EXPERTS), jnp.float32),
            jax.ShapeDtypeStruct((1, EXPERTS), jnp.int32),
            jax.ShapeDtypeStruct((1, 1), jnp.float32),
            jax.ShapeDtypeStruct((1, 1), jnp.float32),
        ],
        scratch_shapes=[
            pltpu.VMEM((1, EXPERTS), jnp.float32),
            pltpu.VMEM((1, EXPERTS), jnp.float32),
            pltpu.VMEM((1, EXPERTS), jnp.float32),
        ],
    )(hs, wg, tril)

    comb = _combine_sc(rp, p0, p1, c0.reshape(EXPERTS))

    disp8 = pl.pallas_call(
        _dispatch_body,
        grid=(NBLK,),
        in_specs=[meta_spec, meta_spec, vec_spec],
        out_specs=[
            pl.BlockSpec((BLK, EXPERTS, CAPACITY), lambda i: (i, 0, 0)),
        ],
        out_shape=[
            jax.ShapeDtypeStruct((SEQ, EXPERTS, CAPACITY), jnp.int8),
        ],
    )(p0, p1, c0)[0]

    return (
        laux.reshape(()),
        rate.reshape(()),
        comb,
        disp8.astype(jnp.bool_),
        cnt.reshape(EXPERTS),
    )


def kernel(hidden_states, wg_weight):
    hs = hidden_states.reshape(-1, HIDDEN).astype(jnp.float32)
    return _run(hs, wg_weight)


# speedup vs baseline: 1.9597x; 1.0240x over previous
import functools

import jax
import jax.numpy as jnp
from jax import lax
from jax.experimental import pallas as pl
from jax.experimental.pallas import tpu as pltpu
from jax.experimental.pallas import tpu_sc as plsc

SEQ = 2048
EXPERTS = 16
HIDDEN = 2048
TOPK = 2
CAPACITY = 256
BLK = 512
NBLK = SEQ // BLK

NWORKERS = 32
TOK_PER_W = SEQ // NWORKERS
CHUNK = 8
NCHUNK = TOK_PER_W // CHUNK
ROW = EXPERTS * CAPACITY
CHUNK_WORDS = CHUNK * ROW


def _inclusive_cumsum_rows(x):
    n = x.shape[0]
    d = 1
    while d < n:
        shifted = jnp.concatenate(
            [jnp.zeros((d, x.shape[1]), x.dtype), x[:-d, :]], axis=0
        )
        x = x + shifted
        d *= 2
    return x


def _routing_body(hs_ref, wg_ref, rp_ref, p0_ref, p1_ref, c0_ref,
                  cnt_ref, laux_ref, rate_ref, offs0, offs1, sumg):
    i = pl.program_id(0)

    @pl.when(i == 0)
    def _init():
        offs0[...] = jnp.zeros_like(offs0)
        offs1[...] = jnp.zeros_like(offs1)
        sumg[...] = jnp.zeros_like(sumg)

    x = hs_ref[...]
    w = wg_ref[...]
    logits = jax.lax.dot_general(
        x, w, (((1,), (1,)), ((), ())), preferred_element_type=jnp.float32
    )

    m = jnp.max(logits, axis=1, keepdims=True)
    ex = jnp.exp(logits - m)
    g = ex / jnp.sum(ex, axis=1, keepdims=True)

    iota = jax.lax.broadcasted_iota(jnp.int32, (BLK, EXPERTS), 1)
    v0 = jnp.max(g, axis=1, keepdims=True)
    idx0 = jnp.min(jnp.where(g == v0, iota, EXPERTS), axis=1, keepdims=True)
    m0 = iota == idx0
    g_ex = jnp.where(m0, -jnp.inf, g)
    v1 = jnp.max(g_ex, axis=1, keepdims=True)
    idx1 = jnp.min(jnp.where(g_ex == v1, iota, EXPERTS), axis=1, keepdims=True)
    m1 = iota == idx1

    gates_s = jnp.maximum(v0 + v1, jnp.finfo(jnp.float32).eps)
    rp_ref[...] = g / gates_s

    m0f = m0.astype(jnp.float32)
    m1f = m1.astype(jnp.float32)
    rows = jax.lax.broadcasted_iota(jnp.int32, (BLK, BLK), 0)
    cols = jax.lax.broadcasted_iota(jnp.int32, (BLK, BLK), 1)
    tril = (cols < rows).astype(jnp.float32)
    exc0 = jax.lax.dot_general(
        tril, m0f, (((1,), (0,)), ((), ())), preferred_element_type=jnp.float32
    )
    exc1 = jax.lax.dot_general(
        tril, m1f, (((1,), (0,)), ((), ())), preferred_element_type=jnp.float32
    )

    p0_ref[...] = jnp.where(m0, offs0[...] + exc0, -1.0)
    p1_ref[...] = jnp.where(m1, offs1[...] + exc1, -1.0)

    tot0 = exc0[BLK - 1 : BLK, :] + m0f[BLK - 1 : BLK, :]
    tot1 = exc1[BLK - 1 : BLK, :] + m1f[BLK - 1 : BLK, :]
    offs0[...] = offs0[...] + tot0
    offs1[...] = offs1[...] + tot1
    sumg[...] = sumg[...] + jnp.sum(g, axis=0, keepdims=True)
    c0_ref[...] = offs0[...]

    @pl.when(i == NBLK - 1)
    def _finish():
        ctot = offs0[...] + offs1[...]
        cnt_ref[...] = ctot.astype(jnp.int32)
        inv_s = 1.0 / SEQ
        laux = (EXPERTS * EXPERTS) * jnp.mean(
            (ctot * inv_s) * (sumg[...] * inv_s)
        )
        laux_ref[0, 0] = laux
        rate_ref[0, 0] = jnp.sum(jnp.minimum(ctot, float(CAPACITY))) / (
            SEQ * TOPK
        )


def _dispatch_body(p0_ref, p1_ref, c0_ref, disp_ref):
    p0 = p0_ref[...]
    p1p = p1_ref[...]
    c0 = c0_ref[...]

    p1 = jnp.where(p1p >= 0.0, p1p + c0, -1.0)
    tp = jnp.maximum(p0, p1)
    valid = jnp.logical_and(tp >= 0.0, tp < float(CAPACITY))
    tpc = jnp.where(valid, tp, -1.0).astype(jnp.int32)

    cap_iota = jax.lax.broadcasted_iota(
        jnp.int32, (BLK, EXPERTS, CAPACITY), 2
    )
    disp_ref[...] = (tpc[:, :, None] == cap_iota).astype(jnp.int8)


def _combine_sc_body(rp_hbm, p0_hbm, p1_hbm, c0_hbm, out_hbm,
                     rp_v, p0_v, p1_v, c0_v, buf0, buf1, buf2,
                     sem0, sem1, sem2):
    wid = lax.axis_index("s") * 2 + lax.axis_index("c")
    base = wid * TOK_PER_W

    pltpu.sync_copy(rp_hbm.at[pl.ds(base, TOK_PER_W)], rp_v)
    pltpu.sync_copy(p0_hbm.at[pl.ds(base, TOK_PER_W)], p0_v)
    pltpu.sync_copy(p1_hbm.at[pl.ds(base, TOK_PER_W)], p1_v)
    pltpu.sync_copy(c0_hbm, c0_v)

    lanes = lax.iota(jnp.int32, 16)
    zf = jnp.zeros((16,), jnp.float32)
    c0r = c0_v[...]
    bufs = [buf0, buf1, buf2]
    sems = [sem0, sem1, sem2]

    ic_list = [lanes + 16 * k for k in range(CAPACITY // 16)]
    ie_list = [jnp.full((16,), e, jnp.int32) for e in range(EXPERTS)]

    def _zero_buf(buf):
        def body(t, _):
            it = jnp.full((16,), t, jnp.int32)
            for e in range(EXPERTS):
                for k in range(CAPACITY // 16):
                    plsc.store_scatter(buf, [it, ie_list[e], ic_list[k]], zf)
            return _
        lax.fori_loop(0, CHUNK, body, 0)

    def _entry(tok):
        p0r = p0_v[tok]
        p1r = p1_v[tok]
        rpr = rp_v[tok]
        p1f = jnp.where(p1r >= 0.0, p1r + c0r, -1.0)
        tp = jnp.maximum(p0r, p1f)
        valid = jnp.logical_and(tp >= 0.0, tp < float(CAPACITY))
        pos = jnp.minimum(jnp.maximum(tp, 0.0), float(CAPACITY - 1))
        return pos.astype(jnp.int32), valid, rpr

    def _build(buf, cidx):
        def body(t, _):
            it = jnp.full((16,), t, jnp.int32)
            ic, msk, rpr = _entry(cidx * CHUNK + t)
            plsc.store_scatter(buf, [it, lanes, ic], rpr, mask=msk)
            return _
        lax.fori_loop(0, CHUNK, body, 0)

    def _clear(buf, cidx):
        def body(t, _):
            it = jnp.full((16,), t, jnp.int32)
            ic, msk, _rpr = _entry(cidx * CHUNK + t)
            plsc.store_scatter(buf, [it, lanes, ic], zf, mask=msk)
            return _
        lax.fori_loop(0, CHUNK, body, 0)

    copies = [None, None, None]
    for cidx in range(NCHUNK):
        b = cidx % 3
        buf = bufs[b]
        if copies[b] is None:
            _zero_buf(buf)
        else:
            copies[b].wait()
            _clear(buf, cidx - 3)
        _build(buf, cidx)
        copies[b] = pltpu.async_copy(
            buf,
            out_hbm.at[pl.ds(base + cidx * CHUNK, CHUNK)],
            sems[b],
        )
    for b in range(3):
        if copies[b] is not None:
            copies[b].wait()


_combine_sc = functools.partial(
    pl.kernel,
    out_type=jax.ShapeDtypeStruct((SEQ, EXPERTS, CAPACITY), jnp.float32),
    mesh=plsc.VectorSubcoreMesh(core_axis_name="c", subcore_axis_name="s"),
    scratch_types=[
        pltpu.VMEM((TOK_PER_W, EXPERTS), jnp.float32),
        pltpu.VMEM((TOK_PER_W, EXPERTS), jnp.float32),
        pltpu.VMEM((TOK_PER_W, EXPERTS), jnp.float32),
        pltpu.VMEM((EXPERTS,), jnp.float32),
        pltpu.VMEM((CHUNK, EXPERTS, CAPACITY), jnp.float32),
        pltpu.VMEM((CHUNK, EXPERTS, CAPACITY), jnp.float32),
        pltpu.VMEM((CHUNK, EXPERTS, CAPACITY), jnp.float32),
        pltpu.SemaphoreType.DMA,
        pltpu.SemaphoreType.DMA,
        pltpu.SemaphoreType.DMA,
    ],
    compiler_params=pltpu.CompilerParams(
        needs_layout_passes=False, skip_device_barrier=True
    ),
)(_combine_sc_body)


@jax.jit
def _run(hs, wg):
    meta_spec = pl.BlockSpec((BLK, EXPERTS), lambda i: (i, 0))
    vec_spec = pl.BlockSpec((1, EXPERTS), lambda i: (0, 0))
    smem_spec = pl.BlockSpec(memory_space=pltpu.SMEM)

    rp, p0, p1, c0, cnt, laux, rate = pl.pallas_call(
        _routing_body,
        grid=(NBLK,),
        in_specs=[
            pl.BlockSpec((BLK, HIDDEN), lambda i: (i, 0)),
            pl.BlockSpec((EXPERTS, HIDDEN), lambda i: (0, 0)),
        ],
        out_specs=[meta_spec, meta_spec, meta_spec, vec_spec, vec_spec,
                   smem_spec, smem_spec],
        out_shape=[
            jax.ShapeDtypeStruct((SEQ, EXPERTS), jnp.float32),
            jax.ShapeDtypeStruct((SEQ, EXPERTS), jnp.float32),
            jax.ShapeDtypeStruct((SEQ, EXPERTS), jnp.float32),
            jax.ShapeDtypeStruct((1, EXPERTS), jnp.float32),
            jax.ShapeDtypeStruct((1, EXPERTS), jnp.int32),
            jax.ShapeDtypeStruct((1, 1), jnp.float32),
            jax.ShapeDtypeStruct((1, 1), jnp.float32),
        ],
        scratch_shapes=[
            pltpu.VMEM((1, EXPERTS), jnp.float32),
            pltpu.VMEM((1, EXPERTS), jnp.float32),
            pltpu.VMEM((1, EXPERTS), jnp.float32),
        ],
    )(hs, wg)

    comb = _combine_sc(rp, p0, p1, c0.reshape(EXPERTS))

    disp8 = pl.pallas_call(
        _dispatch_body,
        grid=(NBLK,),
        in_specs=[meta_spec, meta_spec, vec_spec],
        out_specs=[
            pl.BlockSpec((BLK, EXPERTS, CAPACITY), lambda i: (i, 0, 0)),
        ],
        out_shape=[
            jax.ShapeDtypeStruct((SEQ, EXPERTS, CAPACITY), jnp.int8),
        ],
    )(p0, p1, c0)[0]

    return (
        laux.reshape(()),
        rate.reshape(()),
        comb,
        disp8.astype(jnp.bool_),
        cnt.reshape(EXPERTS),
    )


def kernel(hidden_states, wg_weight):
    hs = hidden_states.reshape(-1, HIDDEN).astype(jnp.float32)
    return _run(hs, wg_weight)


# speedup vs baseline: 2.7278x; 1.3920x over previous
import jax
import jax.numpy as jnp
from jax.experimental import pallas as pl
from jax.experimental.pallas import tpu as pltpu

SEQ = 2048
EXPERTS = 16
HIDDEN = 2048
TOPK = 2
CAPACITY = 256
BLK = 512
NBLK = SEQ // BLK


def _routing_body(hs_ref, wg_ref, rp_ref, p0_ref, p1_ref, c0_ref,
                  cnt_ref, laux_ref, rate_ref, offs0, offs1, sumg):
    i = pl.program_id(0)

    @pl.when(i == 0)
    def _init():
        offs0[...] = jnp.zeros_like(offs0)
        offs1[...] = jnp.zeros_like(offs1)
        sumg[...] = jnp.zeros_like(sumg)

    x = hs_ref[...]
    w = wg_ref[...]
    logits = jax.lax.dot_general(
        x, w, (((1,), (1,)), ((), ())), preferred_element_type=jnp.float32
    )

    m = jnp.max(logits, axis=1, keepdims=True)
    ex = jnp.exp(logits - m)
    g = ex / jnp.sum(ex, axis=1, keepdims=True)

    iota = jax.lax.broadcasted_iota(jnp.int32, (BLK, EXPERTS), 1)
    v0 = jnp.max(g, axis=1, keepdims=True)
    idx0 = jnp.min(jnp.where(g == v0, iota, EXPERTS), axis=1, keepdims=True)
    m0 = iota == idx0
    g_ex = jnp.where(m0, -jnp.inf, g)
    v1 = jnp.max(g_ex, axis=1, keepdims=True)
    idx1 = jnp.min(jnp.where(g_ex == v1, iota, EXPERTS), axis=1, keepdims=True)
    m1 = iota == idx1

    gates_s = jnp.maximum(v0 + v1, jnp.finfo(jnp.float32).eps)
    rp_ref[...] = g / gates_s

    m0f = m0.astype(jnp.float32)
    m1f = m1.astype(jnp.float32)
    rows = jax.lax.broadcasted_iota(jnp.int32, (BLK, BLK), 0)
    cols = jax.lax.broadcasted_iota(jnp.int32, (BLK, BLK), 1)
    tril = (cols < rows).astype(jnp.float32)
    exc0 = jax.lax.dot_general(
        tril, m0f, (((1,), (0,)), ((), ())), preferred_element_type=jnp.float32
    )
    exc1 = jax.lax.dot_general(
        tril, m1f, (((1,), (0,)), ((), ())), preferred_element_type=jnp.float32
    )

    p0_ref[...] = jnp.where(m0, offs0[...] + exc0, -1.0)
    p1_ref[...] = jnp.where(m1, offs1[...] + exc1, -1.0)

    tot0 = exc0[BLK - 1 : BLK, :] + m0f[BLK - 1 : BLK, :]
    tot1 = exc1[BLK - 1 : BLK, :] + m1f[BLK - 1 : BLK, :]
    offs0[...] = offs0[...] + tot0
    offs1[...] = offs1[...] + tot1
    sumg[...] = sumg[...] + jnp.sum(g, axis=0, keepdims=True)
    c0_ref[...] = offs0[...]

    @pl.when(i == NBLK - 1)
    def _finish():
        ctot = offs0[...] + offs1[...]
        cnt_ref[...] = ctot.astype(jnp.int32)
        inv_s = 1.0 / SEQ
        laux = (EXPERTS * EXPERTS) * jnp.mean(
            (ctot * inv_s) * (sumg[...] * inv_s)
        )
        laux_ref[0, 0] = laux
        rate_ref[0, 0] = jnp.sum(jnp.minimum(ctot, float(CAPACITY))) / (
            SEQ * TOPK
        )


def _materialize_body(rp_ref, p0_ref, p1_ref, c0_ref, comb_ref, disp_ref):
    rp = rp_ref[...]
    p0 = p0_ref[...]
    p1p = p1_ref[...]
    c0 = c0_ref[...]

    p1 = jnp.where(p1p >= 0.0, p1p + c0, -1.0)
    tp = jnp.maximum(p0, p1)
    valid = jnp.logical_and(tp >= 0.0, tp < float(CAPACITY))
    tpc = jnp.where(valid, tp, -1.0).astype(jnp.int32)

    cap_iota = jax.lax.broadcasted_iota(
        jnp.int32, (BLK, EXPERTS, CAPACITY), 2
    )
    disp = tpc[:, :, None] == cap_iota
    disp_ref[...] = disp.astype(jnp.int8)
    comb_ref[...] = jnp.where(disp, rp[:, :, None], 0.0)


@jax.jit
def _run(hs, wg):
    meta_spec = pl.BlockSpec((BLK, EXPERTS), lambda i: (i, 0))
    vec_spec = pl.BlockSpec((1, EXPERTS), lambda i: (0, 0))
    smem_spec = pl.BlockSpec(memory_space=pltpu.SMEM)

    rp, p0, p1, c0, cnt, laux, rate = pl.pallas_call(
        _routing_body,
        grid=(NBLK,),
        in_specs=[
            pl.BlockSpec((BLK, HIDDEN), lambda i: (i, 0)),
            pl.BlockSpec((EXPERTS, HIDDEN), lambda i: (0, 0)),
        ],
        out_specs=[meta_spec, meta_spec, meta_spec, vec_spec, vec_spec,
                   smem_spec, smem_spec],
        out_shape=[
            jax.ShapeDtypeStruct((SEQ, EXPERTS), jnp.float32),
            jax.ShapeDtypeStruct((SEQ, EXPERTS), jnp.float32),
            jax.ShapeDtypeStruct((SEQ, EXPERTS), jnp.float32),
            jax.ShapeDtypeStruct((1, EXPERTS), jnp.float32),
            jax.ShapeDtypeStruct((1, EXPERTS), jnp.int32),
            jax.ShapeDtypeStruct((1, 1), jnp.float32),
            jax.ShapeDtypeStruct((1, 1), jnp.float32),
        ],
        scratch_shapes=[
            pltpu.VMEM((1, EXPERTS), jnp.float32),
            pltpu.VMEM((1, EXPERTS), jnp.float32),
            pltpu.VMEM((1, EXPERTS), jnp.float32),
        ],
    )(hs, wg)

    comb, disp8 = pl.pallas_call(
        _materialize_body,
        grid=(NBLK,),
        in_specs=[meta_spec, meta_spec, meta_spec, vec_spec],
        out_specs=[
            pl.BlockSpec((BLK, EXPERTS, CAPACITY), lambda i: (i, 0, 0)),
            pl.BlockSpec((BLK, EXPERTS, CAPACITY), lambda i: (i, 0, 0)),
        ],
        out_shape=[
            jax.ShapeDtypeStruct((SEQ, EXPERTS, CAPACITY), jnp.float32),
            jax.ShapeDtypeStruct((SEQ, EXPERTS, CAPACITY), jnp.int8),
        ],
    )(rp, p0, p1, c0)

    return (
        laux.reshape(()),
        rate.reshape(()),
        comb,
        disp8.astype(jnp.bool_),
        cnt.reshape(EXPERTS),
    )


def kernel(hidden_states, wg_weight):
    hs = hidden_states.reshape(-1, HIDDEN).astype(jnp.float32)
    return _run(hs, wg_weight)


# speedup vs baseline: 2.7347x; 1.0025x over previous
import jax
import jax.numpy as jnp
from jax.experimental import pallas as pl
from jax.experimental.pallas import tpu as pltpu

SEQ = 2048
EXPERTS = 16
HIDDEN = 2048
TOPK = 2
CAPACITY = 256
BLK = 512
NBLK = SEQ // BLK


def _routing_body(hs_ref, wg_ref, rp_ref, p0_ref, p1_ref, c0_ref,
                  cnt_ref, laux_ref, rate_ref, offs0, offs1, sumg, tril_s):
    i = pl.program_id(0)

    @pl.when(i == 0)
    def _init():
        offs0[...] = jnp.zeros_like(offs0)
        offs1[...] = jnp.zeros_like(offs1)
        sumg[...] = jnp.zeros_like(sumg)
        rows = jax.lax.broadcasted_iota(jnp.int32, (BLK, BLK), 0)
        cols = jax.lax.broadcasted_iota(jnp.int32, (BLK, BLK), 1)
        tril_s[...] = (cols < rows).astype(jnp.float32)

    x = hs_ref[...]
    w = wg_ref[...]
    logits = jax.lax.dot_general(
        x, w, (((1,), (1,)), ((), ())), preferred_element_type=jnp.float32
    )

    m = jnp.max(logits, axis=1, keepdims=True)
    ex = jnp.exp(logits - m)
    g = ex / jnp.sum(ex, axis=1, keepdims=True)

    erows = jax.lax.broadcasted_iota(jnp.int32, (EXPERTS, EXPERTS), 0)
    ecols = jax.lax.broadcasted_iota(jnp.int32, (EXPERTS, EXPERTS), 1)
    triu = (erows < ecols).astype(jnp.float32)
    v0 = jnp.max(g, axis=1, keepdims=True)
    cand0 = (g == v0).astype(jnp.float32)
    before0 = jax.lax.dot_general(
        cand0, triu, (((1,), (0,)), ((), ())),
        preferred_element_type=jnp.float32,
    )
    m0 = jnp.logical_and(cand0 > 0.0, before0 == 0.0)
    g_ex = jnp.where(m0, -jnp.inf, g)
    v1 = jnp.max(g_ex, axis=1, keepdims=True)
    cand1 = (g_ex == v1).astype(jnp.float32)
    before1 = jax.lax.dot_general(
        cand1, triu, (((1,), (0,)), ((), ())),
        preferred_element_type=jnp.float32,
    )
    m1 = jnp.logical_and(cand1 > 0.0, before1 == 0.0)

    gates_s = jnp.maximum(v0 + v1, jnp.finfo(jnp.float32).eps)
    rp_ref[...] = g / gates_s

    m0f = m0.astype(jnp.float32)
    m1f = m1.astype(jnp.float32)
    tril = tril_s[...]
    exc0 = jax.lax.dot_general(
        tril, m0f, (((1,), (0,)), ((), ())), preferred_element_type=jnp.float32
    )
    exc1 = jax.lax.dot_general(
        tril, m1f, (((1,), (0,)), ((), ())), preferred_element_type=jnp.float32
    )

    p0_ref[...] = jnp.where(m0, offs0[...] + exc0, -1.0)
    p1_ref[...] = jnp.where(m1, offs1[...] + exc1, -1.0)

    tot0 = exc0[BLK - 1 : BLK, :] + m0f[BLK - 1 : BLK, :]
    tot1 = exc1[BLK - 1 : BLK, :] + m1f[BLK - 1 : BLK, :]
    offs0[...] = offs0[...] + tot0
    offs1[...] = offs1[...] + tot1
    sumg[...] = sumg[...] + jnp.sum(g, axis=0, keepdims=True)
    c0_ref[...] = offs0[...]

    @pl.when(i == NBLK - 1)
    def _finish():
        ctot = offs0[...] + offs1[...]
        cnt_ref[...] = ctot.astype(jnp.int32)
        inv_s = 1.0 / SEQ
        laux = (EXPERTS * EXPERTS) * jnp.mean(
            (ctot * inv_s) * (sumg[...] * inv_s)
        )
        laux_ref[0, 0] = laux
        rate_ref[0, 0] = jnp.sum(jnp.minimum(ctot, float(CAPACITY))) / (
            SEQ * TOPK
        )


def _materialize_body(rp_ref, p0_ref, p1_ref, c0_ref, comb_ref, disp_ref):
    rp = rp_ref[...]
    p0 = p0_ref[...]
    p1p = p1_ref[...]
    c0 = c0_ref[...]

    p1 = jnp.where(p1p >= 0.0, p1p + c0, -1.0)
    tp = jnp.maximum(p0, p1)
    valid = jnp.logical_and(tp >= 0.0, tp < float(CAPACITY))
    tpc = jnp.where(valid, tp, -1.0).astype(jnp.int32)

    cap_iota = jax.lax.broadcasted_iota(
        jnp.int32, (BLK, EXPERTS, CAPACITY), 2
    )
    disp = tpc[:, :, None] == cap_iota
    disp_ref[...] = disp.astype(jnp.int8)
    comb_ref[...] = jnp.where(disp, rp[:, :, None], 0.0)


@jax.jit
def _run(hs, wg):
    meta_spec = pl.BlockSpec((BLK, EXPERTS), lambda i: (i, 0))
    vec_spec = pl.BlockSpec((1, EXPERTS), lambda i: (0, 0))
    smem_spec = pl.BlockSpec(memory_space=pltpu.SMEM)

    rp, p0, p1, c0, cnt, laux, rate = pl.pallas_call(
        _routing_body,
        grid=(NBLK,),
        in_specs=[
            pl.BlockSpec((BLK, HIDDEN), lambda i: (i, 0)),
            pl.BlockSpec((EXPERTS, HIDDEN), lambda i: (0, 0)),
        ],
        out_specs=[meta_spec, meta_spec, meta_spec, vec_spec, vec_spec,
                   smem_spec, smem_spec],
        out_shape=[
            jax.ShapeDtypeStruct((SEQ, EXPERTS), jnp.float32),
            jax.ShapeDtypeStruct((SEQ, EXPERTS), jnp.float32),
            jax.ShapeDtypeStruct((SEQ, EXPERTS), jnp.float32),
            jax.ShapeDtypeStruct((1, EXPERTS), jnp.float32),
            jax.ShapeDtypeStruct((1, EXPERTS), jnp.int32),
            jax.ShapeDtypeStruct((1, 1), jnp.float32),
            jax.ShapeDtypeStruct((1, 1), jnp.float32),
        ],
        scratch_shapes=[
            pltpu.VMEM((1, EXPERTS), jnp.float32),
            pltpu.VMEM((1, EXPERTS), jnp.float32),
            pltpu.VMEM((1, EXPERTS), jnp.float32),
            pltpu.VMEM((BLK, BLK), jnp.float32),
        ],
    )(hs, wg)

    comb, disp8 = pl.pallas_call(
        _materialize_body,
        grid=(NBLK,),
        in_specs=[meta_spec, meta_spec, meta_spec, vec_spec],
        out_specs=[
            pl.BlockSpec((BLK, EXPERTS, CAPACITY), lambda i: (i, 0, 0)),
            pl.BlockSpec((BLK, EXPERTS, CAPACITY), lambda i: (i, 0, 0)),
        ],
        out_shape=[
            jax.ShapeDtypeStruct((SEQ, EXPERTS, CAPACITY), jnp.float32),
            jax.ShapeDtypeStruct((SEQ, EXPERTS, CAPACITY), jnp.int8),
        ],
    )(rp, p0, p1, c0)

    return (
        laux.reshape(()),
        rate.reshape(()),
        comb,
        disp8.astype(jnp.bool_),
        cnt.reshape(EXPERTS),
    )


def kernel(hidden_states, wg_weight):
    hs = hidden_states.reshape(-1, HIDDEN).astype(jnp.float32)
    return _run(hs, wg_weight)
